# baseline (speedup 1.0000x reference)
import jax
import jax.numpy as jnp
from jax import lax
from jax.experimental import pallas as pl


_H = 32
_W = 32
_D = 256
_Q = _H * _W


def _body(tab_ref, o_ref):
    half = lax.rem(pl.program_id(0), 2)
    tab = tab_ref[0]
    q = lax.broadcasted_iota(jnp.int32, (_W, _Q), 1)
    r = lax.broadcasted_iota(jnp.int32, (_W, _Q), 0)
    idx = jnp.where(half == 0, lax.rem(q, _W), lax.div(q, _W))
    sel = (idx == r).astype(jnp.float32)
    o_ref[0] = lax.dot_general(
        tab, sel, (((0,), (0,)), ((), ())),
        preferred_element_type=jnp.float32)


def kernel(x, row_embed, col_embed):
    b, _, h, w = x.shape
    d = row_embed.shape[-1]
    tabs = jnp.stack([col_embed[:w], row_embed[:h]])
    out = pl.pallas_call(
        _body,
        grid=(2 * b,),
        in_specs=[pl.BlockSpec((1, w, d), lambda j: (j % 2, 0, 0))],
        out_specs=pl.BlockSpec((1, d, h * w), lambda j: (j, 0, 0)),
        out_shape=jax.ShapeDtypeStruct((2 * b, d, h * w), jnp.float32),
    )(tabs)
    return out.reshape(b, 2 * d, h, w)
